# baseline (speedup 1.0000x reference)
import jax
import jax.numpy as jnp
from jax.experimental import pallas as pl
from jax.experimental.pallas import tpu as pltpu

N = 8192
BM = 256

_DOT = dict(precision=jax.lax.Precision.DEFAULT,
            preferred_element_type=jnp.float32)


def _lsm(z):
    m = jnp.max(z, axis=1, keepdims=True)
    e = z - m
    return e - jnp.log(jnp.sum(jnp.exp(e), axis=1, keepdims=True))


def _pair_kernel_1(x_ref, aa_ref, ap_ref,
                   w1_ref, b1_ref, w2_ref, b2_ref, wl_ref, bl_ref,
                   oa_ref, lsm_p_ref,
                   s1_ref, s2_ref):
    ph = pl.program_id(0)
    i = pl.program_id(1)

    @pl.when((ph == 0) & (i == 0))
    def _build_s1():
        s1_ref[...] = jnp.dot(x_ref[...], w1_ref[...], **_DOT)

    @pl.when(ph == 0)
    def _layer1():
        za = jnp.dot(aa_ref[...], s1_ref[:, 0:32], **_DOT)
        zp = jnp.dot(ap_ref[...], s1_ref[:, 32:64], **_DOT)
        h = jnp.tanh(jnp.concatenate([za, zp], axis=1) + b1_ref[0:1, :])
        s2_ref[pl.ds(i * BM, BM), :] = jnp.dot(h, w2_ref[...], **_DOT)

    @pl.when(ph == 1)
    def _layer2():
        za = jnp.dot(aa_ref[...], s2_ref[:, 0:16], **_DOT)
        zp = jnp.dot(ap_ref[...], s2_ref[:, 16:32], **_DOT)
        h2 = jnp.tanh(jnp.concatenate([za, zp], axis=1) + b2_ref[0:1, :])
        ob = jnp.dot(h2, wl_ref[...], **_DOT) + bl_ref[0:1, :]
        oa_ref[...] = ob[:, 0:8]
        lsm_p_ref[...] = _lsm(ob[:, 8:16])


def _pair_kernel_2(x_ref, aa_ref, ap_ref,
                   w1_ref, b1_ref, w2_ref, b2_ref, wl_ref, bl_ref,
                   oa1_ref, wf_ref, bf_ref,
                   lsm_f_ref, lsm_p_ref, fused_ref,
                   s1_ref, s2_ref):
    ph = pl.program_id(0)
    i = pl.program_id(1)

    @pl.when((ph == 0) & (i == 0))
    def _build_s1():
        s1_ref[...] = jnp.dot(x_ref[...], w1_ref[...], **_DOT)

    @pl.when(ph == 0)
    def _layer1():
        za = jnp.dot(aa_ref[...], s1_ref[:, 0:32], **_DOT)
        zp = jnp.dot(ap_ref[...], s1_ref[:, 32:64], **_DOT)
        h = jnp.tanh(jnp.concatenate([za, zp], axis=1) + b1_ref[0:1, :])
        s2_ref[pl.ds(i * BM, BM), :] = jnp.dot(h, w2_ref[...], **_DOT)

    @pl.when(ph == 1)
    def _layer2():
        za = jnp.dot(aa_ref[...], s2_ref[:, 0:16], **_DOT)
        zp = jnp.dot(ap_ref[...], s2_ref[:, 16:32], **_DOT)
        h2 = jnp.tanh(jnp.concatenate([za, zp], axis=1) + b2_ref[0:1, :])
        ob = jnp.dot(h2, wl_ref[...], **_DOT) + bl_ref[0:1, :]
        oc = jnp.concatenate([oa1_ref[...], ob[:, 0:8]], axis=1)
        fused = jnp.dot(oc, wf_ref[...], **_DOT) + bf_ref[0:1, :]
        lsm_f_ref[...] = _lsm(fused)
        lsm_p_ref[...] = _lsm(ob[:, 8:16])
        fused_ref[...] = fused


def kernel(x, A1, P1, A2, P2,
           W1_A1, b1_A1, W2_A1, b2_A1, Wl_A1, bl_A1,
           W1_A2, b1_A2, W2_A2, b2_A2, Wl_A2, bl_A2,
           W1_P1, b1_P1, W2_P1, b2_P1, Wl_P1, bl_P1,
           W1_P2, b1_P2, W2_P2, b2_P2, Wl_P2, bl_P2,
           Wf, bf):
    f32 = jnp.float32
    bd = jax.scipy.linalg.block_diag
    r8 = lambda v: jnp.broadcast_to(v[None, :], (8, v.shape[0]))
    cat = jnp.concatenate

    grid = (2, N // BM)
    a_spec = pl.BlockSpec((BM, N), lambda ph, i: (i, 0))
    full = lambda r, c: pl.BlockSpec((r, c), lambda ph, i: (0, 0))
    o_spec = pl.BlockSpec((BM, 8), lambda ph, i: (i, 0))
    scratch = [pltpu.VMEM((N, 64), f32),
               pltpu.VMEM((N, 32), f32)]
    params = pltpu.CompilerParams(
        dimension_semantics=("arbitrary", "arbitrary"))

    o_A1, lsm_p1 = pl.pallas_call(
        _pair_kernel_1,
        grid=grid,
        in_specs=[full(N, 128), a_spec, a_spec,
                  full(128, 64), full(8, 64), full(64, 32), full(8, 32),
                  full(32, 16), full(8, 16)],
        out_specs=[o_spec, o_spec],
        out_shape=[jax.ShapeDtypeStruct((N, 8), f32) for _ in range(2)],
        scratch_shapes=scratch,
        compiler_params=params,
    )(x, A1, P1,
      cat([W1_A1, W1_P1], axis=1), r8(cat([b1_A1, b1_P1])),
      bd(W2_A1, W2_P1), r8(cat([b2_A1, b2_P1])),
      bd(Wl_A1, Wl_P1), r8(cat([bl_A1, bl_P1])))

    lsm_f, lsm_p2, fused = pl.pallas_call(
        _pair_kernel_2,
        grid=grid,
        in_specs=[full(N, 128), a_spec, a_spec,
                  full(128, 64), full(8, 64), full(64, 32), full(8, 32),
                  full(32, 16), full(8, 16),
                  o_spec, full(16, 8), full(8, 8)],
        out_specs=[o_spec, o_spec, o_spec],
        out_shape=[jax.ShapeDtypeStruct((N, 8), f32) for _ in range(3)],
        scratch_shapes=scratch,
        compiler_params=params,
    )(x, A2, P2,
      cat([W1_A2, W1_P2], axis=1), r8(cat([b1_A2, b1_P2])),
      bd(W2_A2, W2_P2), r8(cat([b2_A2, b2_P2])),
      bd(Wl_A2, Wl_P2), r8(cat([bl_A2, bl_P2])),
      o_A1, Wf, r8(bf))

    return (lsm_f, lsm_p1, lsm_p2, fused)


# speedup vs baseline: 1.0217x; 1.0217x over previous
import jax
import jax.numpy as jnp
from jax.experimental import pallas as pl
from jax.experimental.pallas import tpu as pltpu

N = 8192
BM = 128

_DOT = dict(precision=jax.lax.Precision.DEFAULT,
            preferred_element_type=jnp.float32)


def _mega_kernel(x_ref, a1_ref, p1_ref, a2_ref, p2_ref,
                 w1_ref, b1_ref, w2_ref, b2_ref, wl_ref, bl_ref,
                 wf_ref, bf_ref,
                 out_ref,
                 s1_ref, s2_ref):
    ph = pl.program_id(0)
    i = pl.program_id(1)
    arefs = (a1_ref, p1_ref, a2_ref, p2_ref)

    @pl.when((ph == 0) & (i == 0))
    def _build_s1():
        s1_ref[...] = jnp.dot(x_ref[...], w1_ref[...], **_DOT)

    @pl.when(ph == 0)
    def _layer1():
        zs = [jnp.dot(ar[...], s1_ref[:, 32 * p:32 * (p + 1)], **_DOT)
              for p, ar in enumerate(arefs)]
        h = jnp.tanh(jnp.concatenate(zs, axis=1) + b1_ref[0:1, :])
        s2_ref[pl.ds(i * BM, BM), :] = jnp.dot(h, w2_ref[...], **_DOT)

    @pl.when(ph == 1)
    def _layer2():
        zs = [jnp.dot(ar[...], s2_ref[:, 16 * p:16 * (p + 1)], **_DOT)
              for p, ar in enumerate(arefs)]
        h2 = jnp.tanh(jnp.concatenate(zs, axis=1) + b2_ref[0:1, :])
        ob = jnp.dot(h2, wl_ref[...], **_DOT) + bl_ref[0:1, :]
        fused = jnp.dot(ob, wf_ref[...], **_DOT) + bf_ref[0:1, :]

        def lsm(z):
            m = jnp.max(z, axis=1, keepdims=True)
            e = z - m
            return e - jnp.log(jnp.sum(jnp.exp(e), axis=1, keepdims=True))

        out_ref[...] = jnp.concatenate(
            [lsm(fused), lsm(ob[:, 8:16]), lsm(ob[:, 24:32]), fused], axis=1)


def kernel(x, A1, P1, A2, P2,
           W1_A1, b1_A1, W2_A1, b2_A1, Wl_A1, bl_A1,
           W1_A2, b1_A2, W2_A2, b2_A2, Wl_A2, bl_A2,
           W1_P1, b1_P1, W2_P1, b2_P1, Wl_P1, bl_P1,
           W1_P2, b1_P2, W2_P2, b2_P2, Wl_P2, bl_P2,
           Wf, bf):
    f32 = jnp.float32
    W1c = jnp.concatenate([W1_A1, W1_P1, W1_A2, W1_P2], axis=1)
    b1c = jnp.broadcast_to(
        jnp.concatenate([b1_A1, b1_P1, b1_A2, b1_P2])[None, :], (8, 128))
    W2bd = jax.scipy.linalg.block_diag(W2_A1, W2_P1, W2_A2, W2_P2)
    b2c = jnp.broadcast_to(
        jnp.concatenate([b2_A1, b2_P1, b2_A2, b2_P2])[None, :], (8, 64))
    Wlbd = jax.scipy.linalg.block_diag(Wl_A1, Wl_P1, Wl_A2, Wl_P2)
    blc = jnp.broadcast_to(
        jnp.concatenate([bl_A1, bl_P1, bl_A2, bl_P2])[None, :], (8, 32))
    Wg = jnp.zeros((32, 8), f32).at[0:8].set(Wf[0:8]).at[16:24].set(Wf[8:16])
    bfc = jnp.broadcast_to(bf[None, :], (8, 8))

    grid = (2, N // BM)
    a_spec = pl.BlockSpec((BM, N), lambda ph, i: (i, 0))
    full = lambda r, c: pl.BlockSpec((r, c), lambda ph, i: (0, 0))
    o_spec = pl.BlockSpec((BM, 32), lambda ph, i: (i, 0))

    out = pl.pallas_call(
        _mega_kernel,
        grid=grid,
        in_specs=[full(N, 128), a_spec, a_spec, a_spec, a_spec,
                  full(128, 128), full(8, 128), full(128, 64), full(8, 64),
                  full(64, 32), full(8, 32), full(32, 8), full(8, 8)],
        out_specs=o_spec,
        out_shape=jax.ShapeDtypeStruct((N, 32), f32),
        scratch_shapes=[pltpu.VMEM((N, 128), f32),
                        pltpu.VMEM((N, 64), f32)],
        compiler_params=pltpu.CompilerParams(
            dimension_semantics=("arbitrary", "arbitrary")),
    )(x, A1, P1, A2, P2, W1c, b1c, W2bd, b2c, Wlbd, blc, Wg, bfc)

    sl = lambda a, b: jax.lax.slice(out, (0, a), (N, b))
    return (sl(0, 8), sl(8, 16), sl(16, 24), sl(24, 32))
